# Initial kernel scaffold; baseline (speedup 1.0000x reference)
import jax
import jax.numpy as jnp
from jax.experimental import pallas as pl


def kernel(frames):
    raise NotImplementedError("write your pallas kernel here")


# speedup vs baseline: 1.4821x; 1.4821x over previous
import jax
import jax.numpy as jnp
from jax.experimental import pallas as pl
from jax.experimental.pallas import tpu as pltpu

_ALPHA = 4


def _pack_body(off_ref, src_ref, slow_ref, fast_ref):
    fast_ref[...] = src_ref[...]
    off = off_ref[pl.program_id(1)]
    slow_ref[...] = src_ref[:, pl.ds(off, 1)]


def kernel(frames):
    C, T, H, W = frames.shape
    S = T // _ALPHA
    idx = jnp.linspace(0, T - 1, S).astype(jnp.int32)
    offs = idx - _ALPHA * jnp.arange(S, dtype=jnp.int32)
    grid_spec = pltpu.PrefetchScalarGridSpec(
        num_scalar_prefetch=1,
        grid=(C, S),
        in_specs=[pl.BlockSpec((1, _ALPHA, H, W), lambda c, t, off: (c, t, 0, 0))],
        out_specs=[
            pl.BlockSpec((1, 1, H, W), lambda c, t, off: (c, t, 0, 0)),
            pl.BlockSpec((1, _ALPHA, H, W), lambda c, t, off: (c, t, 0, 0)),
        ],
    )
    slow, fast = pl.pallas_call(
        _pack_body,
        grid_spec=grid_spec,
        out_shape=[
            jax.ShapeDtypeStruct((C, S, H, W), frames.dtype),
            jax.ShapeDtypeStruct((C, T, H, W), frames.dtype),
        ],
    )(offs, frames)
    return (slow, fast)
